# baseline (speedup 1.0000x reference)
import functools

import jax
import jax.numpy as jnp
from jax.experimental import pallas as pl

BSZ = 128
NV = 1024
NBLK = NV // BSZ
NE = BSZ * BSZ


def _edge_kernel(src_ref, dst_ref, xsr_ref, ew_ref, ewt_ref,
                 w0a_ref, w0b_ref, w1_ref, w2_ref, w3_ref, w4_ref, w5_ref,
                 w6_ref, w7rep_ref, w7f_ref, b0_ref, b1_ref, b2_ref, b3_ref,
                 b4_ref, b5_ref, b6_ref, b7rep_ref, b7f_ref, sel_ref,
                 newx_ref, deg_ref, acc_ref):
    j = pl.program_id(0)
    i = pl.program_id(1)

    src = src_ref[...]
    dstv = dst_ref[...]
    ew = ew_ref[0]

    ws = (w1_ref[...], w2_ref[...], w3_ref[...], w4_ref[...], w5_ref[...],
          w6_ref[...])
    bs = (b1_ref[...], b2_ref[...], b3_ref[...], b4_ref[...], b5_ref[...],
          b6_ref[...])
    w0a = w0a_ref[...]
    w0b = w0b_ref[...]
    b0 = b0_ref[...]

    def hidden(pre):
        h = jnp.maximum(pre, 0.0)
        for wl, bl in zip(ws, bs):
            h = jnp.maximum(jnp.dot(wl, h) + bl, 0.0)
        return h

    h1 = hidden(jnp.dot(w0a, src) + jnp.dot(w0b, dstv) + b0)
    f1 = jnp.dot(w7f_ref[...], h1) + b7f_ref[...]

    xb = xsr_ref[...]
    msg = jnp.zeros((8, NE), jnp.float32)
    for jj in range(8):
        msg = msg + f1[jj * 8:(jj + 1) * 8] * xb[jj:jj + 1]
    wm = msg * ew

    sel = sel_ref[...]
    both = jnp.concatenate([wm, ew], axis=0)
    cd = jnp.dot(both, sel)
    contrib = cd[:8]
    degc = jnp.broadcast_to(cd[8:9], (8, BSZ))

    @pl.when(i == 0)
    def _init_out():
        newx_ref[...] = contrib
        deg_ref[...] = degc

    @pl.when(i != 0)
    def _acc_out():
        newx_ref[...] += contrib
        deg_ref[...] += degc

    @pl.when((i == 0) & (j == 0))
    def _init_acc():
        acc_ref[...] = jnp.zeros((8, BSZ), jnp.float32)

    @pl.when(i <= j)
    def _orth():
        h2 = hidden(jnp.dot(w0a, dstv) + jnp.dot(w0b, src) + b0)
        w7rep = w7rep_ref[...]
        b7rep = b7rep_ref[...]
        pks = [jnp.zeros((8, NE), jnp.float32) for _ in range(8)]
        for jj in range(8):
            e2rep = (jnp.dot(w7rep[jj * 64:(jj + 1) * 64], h2) +
                     b7rep[jj * 64:(jj + 1) * 64])
            f1j = f1[jj * 8:(jj + 1) * 8]
            for kk in range(8):
                pks[kk] = pks[kk] + f1j * e2rep[kk * 8:(kk + 1) * 8]
        frob8 = jnp.zeros((8, NE), jnp.float32)
        for kk in range(8):
            ecol = (jax.lax.broadcasted_iota(jnp.int32, (8, 1), 0) == kk
                    ).astype(jnp.float32)
            dk = pks[kk] - ecol
            frob8 = frob8 + dk * dk
        per_edge = jnp.sqrt(jnp.sum(frob8, axis=0, keepdims=True))

        ewt = ewt_ref[0]
        li = jax.lax.broadcasted_iota(jnp.int32, (1, NE), 1)
        a = li // BSZ
        b = li - a * BSZ
        s_idx = i * BSZ + a
        d_idx = j * BSZ + b
        hr = (ew > 0) & (ewt > 0) & (s_idx < d_idx)
        sum_c = jnp.sum(jnp.where(hr, per_edge, 0.0))
        cnt_c = jnp.sum(hr.astype(jnp.float32))
        ri = jax.lax.broadcasted_iota(jnp.int32, (8, BSZ), 0)
        ci = jax.lax.broadcasted_iota(jnp.int32, (8, BSZ), 1)
        upd = (jnp.where((ri == 0) & (ci == 0), sum_c, 0.0) +
               jnp.where((ri == 1) & (ci == 0), cnt_c, 0.0))
        acc_ref[...] += upd


@functools.partial(jax.jit, static_argnums=())
def _edge_pass(src_exp, dst_exp, xsr_exp, ew_exp, ewt_exp,
               w0a, w0b, w1t, w2t, w3t, w4t, w5t, w6t, w7rep, w7ft,
               b0c, b1c, b2c, b3c, b4c, b5c, b6c, b7rep, b7fc, sel):
    full = lambda shape: pl.BlockSpec(shape, lambda j, i: (0,) * len(shape))
    grid = (NBLK, NBLK)
    return pl.pallas_call(
        _edge_kernel,
        grid=grid,
        in_specs=[
            pl.BlockSpec((8, NE), lambda j, i: (0, i)),
            pl.BlockSpec((8, NE), lambda j, i: (0, j)),
            pl.BlockSpec((8, NE), lambda j, i: (0, i)),
            pl.BlockSpec((1, 1, NE), lambda j, i: (i * NBLK + j, 0, 0)),
            pl.BlockSpec((1, 1, NE), lambda j, i: (i * NBLK + j, 0, 0)),
            full((16, 8)), full((16, 8)),
            full((16, 16)), full((16, 16)), full((16, 16)),
            full((16, 16)), full((16, 16)), full((16, 16)),
            full((512, 16)), full((64, 16)),
            full((16, 1)), full((16, 1)), full((16, 1)), full((16, 1)),
            full((16, 1)), full((16, 1)), full((16, 1)), full((512, 1)),
            full((64, 1)),
            full((NE, BSZ)),
        ],
        out_specs=[
            pl.BlockSpec((8, BSZ), lambda j, i: (0, j)),
            pl.BlockSpec((8, BSZ), lambda j, i: (0, j)),
            pl.BlockSpec((8, BSZ), lambda j, i: (0, 0)),
        ],
        out_shape=[
            jax.ShapeDtypeStruct((8, NV), jnp.float32),
            jax.ShapeDtypeStruct((8, NV), jnp.float32),
            jax.ShapeDtypeStruct((8, BSZ), jnp.float32),
        ],
    )(src_exp, dst_exp, xsr_exp, ew_exp, ewt_exp,
      w0a, w0b, w1t, w2t, w3t, w4t, w5t, w6t, w7rep, w7ft,
      b0c, b1c, b2c, b3c, b4c, b5c, b6c, b7rep, b7fc, sel)


def kernel(xembed, sembed, ylabel, ylprob, wgraph, idvert, W0, b0, W1, b1,
           W2, b2, W3, b3, W4, b4, W5, b5, W6, b6, W7, b7, Wc, bc):
    f32 = jnp.float32
    st = sembed.T.astype(f32)
    src_exp = jnp.broadcast_to(st.reshape(8, NBLK, BSZ, 1),
                               (8, NBLK, BSZ, BSZ)).reshape(8, NBLK * NE)
    dst_exp = jnp.broadcast_to(st.reshape(8, NBLK, 1, BSZ),
                               (8, NBLK, BSZ, BSZ)).reshape(8, NBLK * NE)
    xt = xembed.T.astype(f32)
    xsr_exp = jnp.broadcast_to(xt.reshape(8, NBLK, BSZ, 1),
                               (8, NBLK, BSZ, BSZ)).reshape(8, NBLK * NE)
    wg = wgraph.astype(f32)
    ew_full = jnp.where(wg > 0, wg, 0.0)
    ew_exp = (ew_full.reshape(NBLK, BSZ, NBLK, BSZ)
              .transpose(0, 2, 1, 3).reshape(NBLK * NBLK, 1, NE))
    ewt_full = jnp.where(wg.T > 0, wg.T, 0.0)
    ewt_exp = (ewt_full.reshape(NBLK, BSZ, NBLK, BSZ)
               .transpose(0, 2, 1, 3).reshape(NBLK * NBLK, 1, NE))
    sel = jnp.tile(jnp.eye(BSZ, dtype=f32), (BSZ, 1))

    w7t = W7.T.astype(f32)
    w7ft = w7t.reshape(8, 8, 16).transpose(1, 0, 2).reshape(64, 16)
    b7c = b7.reshape(-1, 1).astype(f32)
    b7fc = b7c.reshape(8, 8, 1).transpose(1, 0, 2).reshape(64, 1)
    w7rep = jnp.broadcast_to(w7t.reshape(64, 1, 16),
                             (64, 8, 16)).reshape(512, 16)
    b7rep = jnp.broadcast_to(b7c.reshape(64, 1, 1),
                             (64, 8, 1)).reshape(512, 1)
    new_xt, deg8, acc = _edge_pass(
        src_exp, dst_exp, xsr_exp, ew_exp, ewt_exp,
        W0[:8].T.astype(f32), W0[8:].T.astype(f32),
        W1.T.astype(f32), W2.T.astype(f32), W3.T.astype(f32),
        W4.T.astype(f32), W5.T.astype(f32), W6.T.astype(f32),
        w7rep, w7ft,
        b0.reshape(-1, 1).astype(f32), b1.reshape(-1, 1).astype(f32),
        b2.reshape(-1, 1).astype(f32), b3.reshape(-1, 1).astype(f32),
        b4.reshape(-1, 1).astype(f32), b5.reshape(-1, 1).astype(f32),
        b6.reshape(-1, 1).astype(f32), b7rep, b7fc,
        sel)

    new_x = new_xt.T
    deg = deg8[0]
    sum_rev = acc[0, 0]
    cnt_rev = acc[1, 0]
    loss_orth = jnp.where(cnt_rev > 0, sum_rev / jnp.maximum(cnt_rev, 1.0),
                          jnp.float32(0.0))
    loss_cons = jnp.float32(0.0)

    denom = jnp.where(deg > 0, deg, 1.0)
    xmaped = jnp.where((deg > 0)[:, None], new_x / denom[:, None], new_x)
    loss_smap = jnp.mean((xmaped - xembed) ** 2) * 8

    glog = jax.nn.log_softmax(xmaped[idvert] @ Wc + bc, axis=1)
    yl = ylprob[idvert]
    kl = jnp.sum(jnp.exp(yl) * (yl - glog), axis=1)
    loss_lbpr = jnp.mean(kl)
    ypred = jnp.argmax(glog, axis=1)
    loss_accs = jnp.mean((ypred == ylabel[idvert]).astype(jnp.float32))
    return (loss_orth, loss_cons, loss_smap, loss_lbpr, loss_accs)


# speedup vs baseline: 1.0885x; 1.0885x over previous
import functools

import jax
import jax.numpy as jnp
from jax.experimental import pallas as pl

BSZ = 128
NV = 1024
NBLK = NV // BSZ
NE = BSZ * BSZ


def _edge_kernel(src_ref, dst_ref, xsr_ref, ew_ref, ewt_ref,
                 w0a_ref, w0b_ref, w1_ref, w2_ref, w3_ref, w4_ref, w5_ref,
                 w6_ref, w7rep_ref, w7f_ref, b0_ref, b1_ref, b2_ref, b3_ref,
                 b4_ref, b5_ref, b6_ref, b7rep_ref, b7f_ref, sel_ref,
                 newx_ref, deg_ref, acc_ref):
    j = pl.program_id(0)
    i = pl.program_id(1)

    src = src_ref[...]
    dstv = dst_ref[...]
    ew = ew_ref[0]

    ws = (w1_ref[...], w2_ref[...], w3_ref[...], w4_ref[...], w5_ref[...],
          w6_ref[...])
    bs = (b1_ref[...], b2_ref[...], b3_ref[...], b4_ref[...], b5_ref[...],
          b6_ref[...])
    w0a = w0a_ref[...]
    w0b = w0b_ref[...]
    b0 = b0_ref[...]

    def hidden(pre):
        h = jnp.maximum(pre, 0.0)
        for wl, bl in zip(ws, bs):
            h = jnp.maximum(jnp.dot(wl, h) + bl, 0.0)
        return h

    h1 = hidden(jnp.dot(w0a, src) + jnp.dot(w0b, dstv) + b0)
    f1 = jnp.dot(w7f_ref[...], h1) + b7f_ref[...]

    xb = xsr_ref[...]
    msg = jnp.zeros((8, NE), jnp.float32)
    for jj in range(8):
        msg = msg + f1[jj * 8:(jj + 1) * 8] * xb[jj:jj + 1]
    wm = msg * ew

    sel = sel_ref[...]
    both = jnp.concatenate([wm, ew], axis=0)
    cd = jnp.dot(both, sel)
    contrib = cd[:8]
    degc = jnp.broadcast_to(cd[8:9], (8, BSZ))

    @pl.when(i == 0)
    def _init_out():
        newx_ref[...] = contrib
        deg_ref[...] = degc

    @pl.when(i != 0)
    def _acc_out():
        newx_ref[...] += contrib
        deg_ref[...] += degc

    @pl.when((i == 0) & (j == 0))
    def _init_acc():
        acc_ref[...] = jnp.zeros((8, BSZ), jnp.float32)

    @pl.when(i <= j)
    def _orth():
        h2 = hidden(jnp.dot(w0a, dstv) + jnp.dot(w0b, src) + b0)
        e2 = jnp.dot(w7rep_ref[...], h2) + b7rep_ref[...]
        frob8 = jnp.zeros((8, NE), jnp.float32)
        for kk in range(8):
            pk = jnp.zeros((8, NE), jnp.float32)
            for jj in range(8):
                pk = pk + (f1[jj * 8:(jj + 1) * 8] *
                           e2[jj * 8 + kk:jj * 8 + kk + 1])
            ecol = (jax.lax.broadcasted_iota(jnp.int32, (8, 1), 0) == kk
                    ).astype(jnp.float32)
            dk = pk - ecol
            frob8 = frob8 + dk * dk
        per_edge = jnp.sqrt(jnp.sum(frob8, axis=0, keepdims=True))

        ewt = ewt_ref[0]
        li = jax.lax.broadcasted_iota(jnp.int32, (1, NE), 1)
        a = li // BSZ
        b = li - a * BSZ
        s_idx = i * BSZ + a
        d_idx = j * BSZ + b
        hr = (ew > 0) & (ewt > 0) & (s_idx < d_idx)
        sum_c = jnp.sum(jnp.where(hr, per_edge, 0.0))
        cnt_c = jnp.sum(hr.astype(jnp.float32))
        ri = jax.lax.broadcasted_iota(jnp.int32, (8, BSZ), 0)
        ci = jax.lax.broadcasted_iota(jnp.int32, (8, BSZ), 1)
        upd = (jnp.where((ri == 0) & (ci == 0), sum_c, 0.0) +
               jnp.where((ri == 1) & (ci == 0), cnt_c, 0.0))
        acc_ref[...] += upd


@functools.partial(jax.jit, static_argnums=())
def _edge_pass(src_exp, dst_exp, xsr_exp, ew_exp, ewt_exp,
               w0a, w0b, w1t, w2t, w3t, w4t, w5t, w6t, w7rep, w7ft,
               b0c, b1c, b2c, b3c, b4c, b5c, b6c, b7rep, b7fc, sel):
    full = lambda shape: pl.BlockSpec(shape, lambda j, i: (0,) * len(shape))
    grid = (NBLK, NBLK)
    return pl.pallas_call(
        _edge_kernel,
        grid=grid,
        in_specs=[
            pl.BlockSpec((8, NE), lambda j, i: (0, i)),
            pl.BlockSpec((8, NE), lambda j, i: (0, j)),
            pl.BlockSpec((8, NE), lambda j, i: (0, i)),
            pl.BlockSpec((1, 1, NE), lambda j, i: (i * NBLK + j, 0, 0)),
            pl.BlockSpec((1, 1, NE), lambda j, i: (i * NBLK + j, 0, 0)),
            full((16, 8)), full((16, 8)),
            full((16, 16)), full((16, 16)), full((16, 16)),
            full((16, 16)), full((16, 16)), full((16, 16)),
            full((64, 16)), full((64, 16)),
            full((16, 1)), full((16, 1)), full((16, 1)), full((16, 1)),
            full((16, 1)), full((16, 1)), full((16, 1)), full((64, 1)),
            full((64, 1)),
            full((NE, BSZ)),
        ],
        out_specs=[
            pl.BlockSpec((8, BSZ), lambda j, i: (0, j)),
            pl.BlockSpec((8, BSZ), lambda j, i: (0, j)),
            pl.BlockSpec((8, BSZ), lambda j, i: (0, 0)),
        ],
        out_shape=[
            jax.ShapeDtypeStruct((8, NV), jnp.float32),
            jax.ShapeDtypeStruct((8, NV), jnp.float32),
            jax.ShapeDtypeStruct((8, BSZ), jnp.float32),
        ],
    )(src_exp, dst_exp, xsr_exp, ew_exp, ewt_exp,
      w0a, w0b, w1t, w2t, w3t, w4t, w5t, w6t, w7rep, w7ft,
      b0c, b1c, b2c, b3c, b4c, b5c, b6c, b7rep, b7fc, sel)


def kernel(xembed, sembed, ylabel, ylprob, wgraph, idvert, W0, b0, W1, b1,
           W2, b2, W3, b3, W4, b4, W5, b5, W6, b6, W7, b7, Wc, bc):
    f32 = jnp.float32
    st = sembed.T.astype(f32)
    src_exp = jnp.broadcast_to(st.reshape(8, NBLK, BSZ, 1),
                               (8, NBLK, BSZ, BSZ)).reshape(8, NBLK * NE)
    dst_exp = jnp.broadcast_to(st.reshape(8, NBLK, 1, BSZ),
                               (8, NBLK, BSZ, BSZ)).reshape(8, NBLK * NE)
    xt = xembed.T.astype(f32)
    xsr_exp = jnp.broadcast_to(xt.reshape(8, NBLK, BSZ, 1),
                               (8, NBLK, BSZ, BSZ)).reshape(8, NBLK * NE)
    wg = wgraph.astype(f32)
    ew_full = jnp.where(wg > 0, wg, 0.0)
    ew_exp = (ew_full.reshape(NBLK, BSZ, NBLK, BSZ)
              .transpose(0, 2, 1, 3).reshape(NBLK * NBLK, 1, NE))
    ewt_full = jnp.where(wg.T > 0, wg.T, 0.0)
    ewt_exp = (ewt_full.reshape(NBLK, BSZ, NBLK, BSZ)
               .transpose(0, 2, 1, 3).reshape(NBLK * NBLK, 1, NE))
    sel = jnp.tile(jnp.eye(BSZ, dtype=f32), (BSZ, 1))

    w7t = W7.T.astype(f32)
    w7ft = w7t.reshape(8, 8, 16).transpose(1, 0, 2).reshape(64, 16)
    b7c = b7.reshape(-1, 1).astype(f32)
    b7fc = b7c.reshape(8, 8, 1).transpose(1, 0, 2).reshape(64, 1)
    new_xt, deg8, acc = _edge_pass(
        src_exp, dst_exp, xsr_exp, ew_exp, ewt_exp,
        W0[:8].T.astype(f32), W0[8:].T.astype(f32),
        W1.T.astype(f32), W2.T.astype(f32), W3.T.astype(f32),
        W4.T.astype(f32), W5.T.astype(f32), W6.T.astype(f32),
        w7t, w7ft,
        b0.reshape(-1, 1).astype(f32), b1.reshape(-1, 1).astype(f32),
        b2.reshape(-1, 1).astype(f32), b3.reshape(-1, 1).astype(f32),
        b4.reshape(-1, 1).astype(f32), b5.reshape(-1, 1).astype(f32),
        b6.reshape(-1, 1).astype(f32), b7c, b7fc,
        sel)

    new_x = new_xt.T
    deg = deg8[0]
    sum_rev = acc[0, 0]
    cnt_rev = acc[1, 0]
    loss_orth = jnp.where(cnt_rev > 0, sum_rev / jnp.maximum(cnt_rev, 1.0),
                          jnp.float32(0.0))
    loss_cons = jnp.float32(0.0)

    denom = jnp.where(deg > 0, deg, 1.0)
    xmaped = jnp.where((deg > 0)[:, None], new_x / denom[:, None], new_x)
    loss_smap = jnp.mean((xmaped - xembed) ** 2) * 8

    glog = jax.nn.log_softmax(xmaped[idvert] @ Wc + bc, axis=1)
    yl = ylprob[idvert]
    kl = jnp.sum(jnp.exp(yl) * (yl - glog), axis=1)
    loss_lbpr = jnp.mean(kl)
    ypred = jnp.argmax(glog, axis=1)
    loss_accs = jnp.mean((ypred == ylabel[idvert]).astype(jnp.float32))
    return (loss_orth, loss_cons, loss_smap, loss_lbpr, loss_accs)
